# baseline (speedup 1.0000x reference)
import jax
import jax.numpy as jnp
from jax.experimental import pallas as pl
from jax.experimental.pallas import tpu as pltpu

_N, _D, _H = 10000, 128, 16
_BM = 200
_NB = _N // _BM
_BC = 400
_NC = _N // _BC


def _xw_kernel(x_ref, w_ref, out_ref):
    out_ref[...] = jnp.dot(x_ref[...], w_ref[...],
                           preferred_element_type=jnp.float32)


def _mega_kernel(adj_ref, xw_ref, wfc_ref, aff_ref,
                 mu_ref, lv_ref, xrec_ref, rec_ref,
                 z_s):
    s = pl.program_id(0)

    @pl.when(s < _NB)
    def _gc_phase():
        t = jnp.dot(adj_ref[...], xw_ref[...],
                    preferred_element_type=jnp.float32)
        t = jnp.where(t >= 0, t, 0.01 * t)
        mu = t[:, :_H]
        mu_ref[...] = mu
        lv_ref[...] = t[:, _H:]
        z_s[pl.ds(s * _BM, _BM), :] = mu
        h = jnp.dot(mu, wfc_ref[...], preferred_element_type=jnp.float32)
        xrec_ref[...] = h * aff_ref[0:1, :] + aff_ref[1:2, :]

    @pl.when(s >= _NB)
    def _ip_phase():
        zb = z_s[pl.ds((s - _NB) * _BC, _BC), :]
        rec_ref[...] = jax.lax.dot_general(
            zb, z_s[...], (((1,), (1,)), ((), ())),
            preferred_element_type=jnp.float32)


def kernel(x, adj, W1, W2, Wfc, bfc, gamma, beta, running_mean, running_var):
    n, d = x.shape
    h = W1.shape[1]

    wcat = jnp.concatenate([W1, W2], axis=1)
    scale = gamma * jax.lax.rsqrt(running_var + 1e-5)
    shift = (bfc - running_mean) * scale + beta
    aff = jnp.stack([scale, shift], axis=0)

    xw = pl.pallas_call(
        _xw_kernel,
        grid=(1,),
        in_specs=[
            pl.BlockSpec((n, d), lambda i: (0, 0)),
            pl.BlockSpec((d, 2 * h), lambda i: (0, 0)),
        ],
        out_specs=pl.BlockSpec((n, 2 * h), lambda i: (0, 0)),
        out_shape=jax.ShapeDtypeStruct((n, 2 * h), jnp.float32),
    )(x, wcat)

    gc_idx = lambda s: (jnp.clip(s, 0, _NB - 1), 0)
    ip_idx = lambda s: (jnp.clip(s - _NB, 0, _NC - 1), 0)

    mu, logvar, x_rec, adj_rec = pl.pallas_call(
        _mega_kernel,
        grid=(_NB + _NC,),
        in_specs=[
            pl.BlockSpec((_BM, n), gc_idx),
            pl.BlockSpec((n, 2 * h), lambda s: (0, 0)),
            pl.BlockSpec((h, d), lambda s: (0, 0)),
            pl.BlockSpec((2, d), lambda s: (0, 0)),
        ],
        out_specs=[
            pl.BlockSpec((_BM, h), gc_idx),
            pl.BlockSpec((_BM, h), gc_idx),
            pl.BlockSpec((_BM, d), gc_idx),
            pl.BlockSpec((_BC, n), ip_idx),
        ],
        out_shape=[
            jax.ShapeDtypeStruct((n, h), jnp.float32),
            jax.ShapeDtypeStruct((n, h), jnp.float32),
            jax.ShapeDtypeStruct((n, d), jnp.float32),
            jax.ShapeDtypeStruct((n, n), jnp.float32),
        ],
        scratch_shapes=[
            pltpu.VMEM((n, h), jnp.float32),
        ],
    )(adj, xw, Wfc, aff)

    z = mu
    return (adj_rec, mu, logvar, z, x_rec)


# speedup vs baseline: 1.0209x; 1.0209x over previous
import jax
import jax.numpy as jnp
from jax.experimental import pallas as pl
from jax.experimental.pallas import tpu as pltpu

_N, _D, _H = 10000, 128, 16
_BM = 400
_NB = _N // _BM
_BC = 200
_NC = _N // _BC


def _xwt_kernel(x_ref, w_ref, out_ref):
    out_ref[...] = jnp.dot(x_ref[...], w_ref[...],
                           preferred_element_type=jnp.float32).T


def _mega_kernel(adj_ref, xwt_ref, wfc_ref, aff_ref,
                 mu_ref, lv_ref, xrec_ref, rec_ref,
                 z_s):
    s = pl.program_id(0)

    @pl.when(s < _NB)
    def _gc_phase():
        t = jax.lax.dot_general(
            adj_ref[...], xwt_ref[...], (((1,), (1,)), ((), ())),
            preferred_element_type=jnp.float32)
        t = jnp.where(t >= 0, t, 0.01 * t)
        mu = t[:, :_H]
        mu_ref[...] = mu
        lv_ref[...] = t[:, _H:]
        z_s[pl.ds(s * _BM, _BM), :] = mu
        h = jnp.dot(mu, wfc_ref[...], preferred_element_type=jnp.float32)
        xrec_ref[...] = h * aff_ref[0:1, :] + aff_ref[1:2, :]

    @pl.when(s >= _NB)
    def _ip_phase():
        zb = z_s[pl.ds((s - _NB) * _BC, _BC), :]
        rec_ref[...] = jax.lax.dot_general(
            zb, z_s[...], (((1,), (1,)), ((), ())),
            preferred_element_type=jnp.float32)


def kernel(x, adj, W1, W2, Wfc, bfc, gamma, beta, running_mean, running_var):
    n, d = x.shape
    h = W1.shape[1]

    wcat = jnp.concatenate([W1, W2], axis=1)
    scale = gamma * jax.lax.rsqrt(running_var + 1e-5)
    shift = (bfc - running_mean) * scale + beta
    aff = jnp.stack([scale, shift], axis=0)

    xwt = pl.pallas_call(
        _xwt_kernel,
        grid=(1,),
        in_specs=[
            pl.BlockSpec((n, d), lambda i: (0, 0)),
            pl.BlockSpec((d, 2 * h), lambda i: (0, 0)),
        ],
        out_specs=pl.BlockSpec((2 * h, n), lambda i: (0, 0)),
        out_shape=jax.ShapeDtypeStruct((2 * h, n), jnp.float32),
    )(x, wcat)

    gc_idx = lambda s: (jnp.clip(s, 0, _NB - 1), 0)
    ip_idx = lambda s: (jnp.clip(s - _NB, 0, _NC - 1), 0)

    mu, logvar, x_rec, adj_rec = pl.pallas_call(
        _mega_kernel,
        grid=(_NB + _NC,),
        in_specs=[
            pl.BlockSpec((_BM, n), gc_idx),
            pl.BlockSpec((2 * h, n), lambda s: (0, 0)),
            pl.BlockSpec((h, d), lambda s: (0, 0)),
            pl.BlockSpec((2, d), lambda s: (0, 0)),
        ],
        out_specs=[
            pl.BlockSpec((_BM, h), gc_idx),
            pl.BlockSpec((_BM, h), gc_idx),
            pl.BlockSpec((_BM, d), gc_idx),
            pl.BlockSpec((_BC, n), ip_idx),
        ],
        out_shape=[
            jax.ShapeDtypeStruct((n, h), jnp.float32),
            jax.ShapeDtypeStruct((n, h), jnp.float32),
            jax.ShapeDtypeStruct((n, d), jnp.float32),
            jax.ShapeDtypeStruct((n, n), jnp.float32),
        ],
        scratch_shapes=[
            pltpu.VMEM((n, h), jnp.float32),
        ],
    )(adj, xwt, Wfc, aff)

    z = mu
    return (adj_rec, mu, logvar, z, x_rec)
